# baseline (speedup 1.0000x reference)
import jax
import jax.numpy as jnp
from jax import lax
from jax.experimental import pallas as pl
from jax.experimental.pallas import tpu as pltpu
from jax.experimental.pallas import tpu_sc as plsc

_B = 128
_N = 4096
_G = _N // _B
_T = _G * (_G + 1) // 2


def _sc_body(nv_hbm, np_hbm, rp_hbm, uc_hbm, u_hbm, out_hbm,
             nvv, npv, rpv, ucv, ublk, dbuf, mbuf):
    info = plsc.get_sparse_core_info()
    nc, ns = info.num_cores, info.num_subcores
    nw = nc * ns
    wid = lax.axis_index("s") * nc + lax.axis_index("c")

    pltpu.sync_copy(nv_hbm, nvv)
    pltpu.sync_copy(np_hbm, npv)
    pltpu.sync_copy(rp_hbm, rpv)
    pltpu.sync_copy(uc_hbm, ucv)

    iota = lax.iota(jnp.int32, 16)
    iotaf = iota.astype(jnp.float32)

    npvec = npv[...]
    nvvec = nvv[...]
    rpvec = rpv[...]
    total = npvec[0]
    for k in range(1, 16):
        total = total + npvec[k]
    u0 = ucv[...][0]
    thresh = u0 * total
    idx = jnp.int32(0)
    csums = []
    acc = None
    for k in range(16):
        acc = npvec[k] if acc is None else acc + npvec[k]
        csums.append(acc)
    for k in range(15, -1, -1):
        idx = jnp.where(csums[k] >= thresh, jnp.int32(k), idx)
    n_nodes = jnp.float32(0.0)
    r = jnp.float32(0.0)
    for k in range(16):
        n_nodes = jnp.where(idx == k, nvvec[k], n_nodes)
        r = jnp.where(idx == k, rpvec[k], r)

    nchunks = _B // 16
    qmax = (_T + nw - 1) // nw

    def pair_step(q, carry):
        p = wid + q * nw

        @pl.when(p < _T)
        def _():
            bi = jnp.int32(0)
            for i in range(1, _G):
                start_i = i * _G - i * (i - 1) // 2
                bi = bi + jnp.where(p >= start_i, jnp.int32(1), jnp.int32(0))
            base = lax.shift_right_logical(bi * (2 * _G + 1 - bi),
                                           jnp.int32(1))
            bj = bi + (p - base)
            r0 = bi * _B
            c0 = bj * _B

            pltpu.sync_copy(u_hbm.at[pl.ds(r0, _B), pl.ds(c0, _B)], ublk)

            r0f = r0.astype(jnp.float32)
            c0f = c0.astype(jnp.float32)

            @pl.when(bi != bj)
            def _():
                def row_step(a, rcarry):
                    rf = r0f + a.astype(jnp.float32)
                    r_eff = jnp.where(rf < n_nodes, r, -1.0)
                    for c in range(nchunks):
                        colf = c0f + (c * 16) + iotaf
                        uvec = ublk[a, pl.ds(c * 16, 16)]
                        li = jnp.where((uvec <= r_eff) & (colf < n_nodes),
                                       1, 0).astype(jnp.int32)
                        dbuf[a, pl.ds(c * 16, 16)] = li
                    rmf = c0f + a.astype(jnp.float32)
                    rm_eff = jnp.where(rmf < n_nodes, r, -1.0)
                    acol = jnp.full((16,), a, jnp.int32)
                    for c in range(nchunks):
                        colmf = r0f + (c * 16) + iotaf
                        uvec = plsc.load_gather(ublk, [c * 16 + iota, acol])
                        li = jnp.where((uvec <= rm_eff) & (colmf < n_nodes),
                                       1, 0).astype(jnp.int32)
                        mbuf[a, pl.ds(c * 16, 16)] = li
                    return rcarry

                lax.fori_loop(0, _B, row_step, 0, unroll=2)
                pltpu.sync_copy(dbuf, out_hbm.at[pl.ds(r0, _B), pl.ds(c0, _B)])
                pltpu.sync_copy(mbuf, out_hbm.at[pl.ds(c0, _B), pl.ds(r0, _B)])

            @pl.when(bi == bj)
            def _():
                def row_step(a, rcarry):
                    rf = r0f + a.astype(jnp.float32)
                    r_eff = jnp.where(rf < n_nodes, r, -1.0)
                    acol = jnp.full((16,), a, jnp.int32)
                    for c in range(nchunks):
                        colf = c0f + (c * 16) + iotaf
                        up = ublk[a, pl.ds(c * 16, 16)]
                        lo = plsc.load_gather(ublk, [c * 16 + iota, acol])
                        hit = ((up <= r_eff) & (colf > rf)) | (
                            (lo <= r_eff) & (colf < rf) & (rf < n_nodes))
                        li = jnp.where(hit & (colf < n_nodes), 1, 0
                                       ).astype(jnp.int32)
                        dbuf[a, pl.ds(c * 16, 16)] = li
                    return rcarry

                lax.fori_loop(0, _B, row_step, 0, unroll=2)
                pltpu.sync_copy(dbuf, out_hbm.at[pl.ds(r0, _B), pl.ds(c0, _B)])

        return carry

    lax.fori_loop(0, qmax, pair_step, 0)


def kernel(N_values, N_probs, r_probs, u_cat, u_links):
    mesh = plsc.VectorSubcoreMesh(core_axis_name="c", subcore_axis_name="s")
    sc = pl.kernel(
        _sc_body,
        mesh=mesh,
        compiler_params=pltpu.CompilerParams(needs_layout_passes=False),
        out_type=jax.ShapeDtypeStruct((_N, _N), jnp.int32),
        scratch_types=[
            pltpu.VMEM((16,), jnp.float32),
            pltpu.VMEM((16,), jnp.float32),
            pltpu.VMEM((16,), jnp.float32),
            pltpu.VMEM((16,), jnp.float32),
            pltpu.VMEM((_B, _B), jnp.float32),
            pltpu.VMEM((_B, _B), jnp.int32),
            pltpu.VMEM((_B, _B), jnp.int32),
        ],
    )
    return sc(
        N_values,
        N_probs,
        r_probs,
        jnp.broadcast_to(u_cat, (16,)),
        u_links,
    )


# speedup vs baseline: 1.6320x; 1.6320x over previous
import jax
import jax.numpy as jnp
from jax import lax
from jax.experimental import pallas as pl
from jax.experimental.pallas import tpu as pltpu
from jax.experimental.pallas import tpu_sc as plsc

_B = 128
_N = 4096
_G = _N // _B
_T = _G * (_G + 1) // 2


def _sc_body(nv_hbm, np_hbm, rp_hbm, uc_hbm, u_hbm, out_hbm,
             nvv, npv, rpv, ucv, ublk, dbuf, mbuf):
    info = plsc.get_sparse_core_info()
    nc, ns = info.num_cores, info.num_subcores
    nw = nc * ns
    wid = lax.axis_index("s") * nc + lax.axis_index("c")

    pltpu.sync_copy(nv_hbm, nvv)
    pltpu.sync_copy(np_hbm, npv)
    pltpu.sync_copy(rp_hbm, rpv)
    pltpu.sync_copy(uc_hbm, ucv)

    iota = lax.iota(jnp.int32, 16)
    iotaf = iota.astype(jnp.float32)

    npvec = npv[...]
    nvvec = nvv[...]
    rpvec = rpv[...]
    total = npvec[0]
    for k in range(1, 16):
        total = total + npvec[k]
    u0 = ucv[...][0]
    thresh = u0 * total
    idx = jnp.int32(0)
    csums = []
    acc = None
    for k in range(16):
        acc = npvec[k] if acc is None else acc + npvec[k]
        csums.append(acc)
    for k in range(15, -1, -1):
        idx = jnp.where(csums[k] >= thresh, jnp.int32(k), idx)
    n_nodes = jnp.float32(0.0)
    r = jnp.float32(0.0)
    for k in range(16):
        n_nodes = jnp.where(idx == k, nvvec[k], n_nodes)
        r = jnp.where(idx == k, rpvec[k], r)

    nchunks = _B // 16
    qmax = (_T + nw - 1) // nw

    def pair_step(q, carry):
        p = wid + q * nw

        @pl.when(p < _T)
        def _():
            bi = jnp.int32(0)
            for i in range(1, _G):
                start_i = i * _G - i * (i - 1) // 2
                bi = bi + jnp.where(p >= start_i, jnp.int32(1), jnp.int32(0))
            base = lax.shift_right_logical(bi * (2 * _G + 1 - bi),
                                           jnp.int32(1))
            bj = bi + (p - base)
            r0 = bi * _B
            c0 = bj * _B

            pltpu.sync_copy(u_hbm.at[pl.ds(r0, _B), pl.ds(c0, _B)], ublk)

            r0f = r0.astype(jnp.float32)
            c0f = c0.astype(jnp.float32)

            @pl.when(bi != bj)
            def _():
                def row_step(a, rcarry):
                    rf = r0f + a.astype(jnp.float32)
                    r_eff = jnp.where(rf < n_nodes, r, -1.0)
                    acol = jnp.full((16,), a, jnp.int32)
                    for c in range(nchunks):
                        colf = c0f + (c * 16) + iotaf
                        uvec = ublk[a, pl.ds(c * 16, 16)]
                        li = jnp.where((uvec <= r_eff) & (colf < n_nodes),
                                       1, 0).astype(jnp.int32)
                        dbuf[a, pl.ds(c * 16, 16)] = li
                        plsc.store_scatter(mbuf, [c * 16 + iota, acol], li)
                    return rcarry

                lax.fori_loop(0, _B, row_step, 0, unroll=4)
                pltpu.sync_copy(dbuf, out_hbm.at[pl.ds(r0, _B), pl.ds(c0, _B)])
                pltpu.sync_copy(mbuf, out_hbm.at[pl.ds(c0, _B), pl.ds(r0, _B)])

            @pl.when(bi == bj)
            def _():
                def row_step(a, rcarry):
                    rf = r0f + a.astype(jnp.float32)
                    r_eff = jnp.where(rf < n_nodes, r, -1.0)
                    acol = jnp.full((16,), a, jnp.int32)
                    for c in range(nchunks):
                        colf = c0f + (c * 16) + iotaf
                        up = ublk[a, pl.ds(c * 16, 16)]
                        lo = plsc.load_gather(ublk, [c * 16 + iota, acol])
                        hit = ((up <= r_eff) & (colf > rf)) | (
                            (lo <= r_eff) & (colf < rf) & (rf < n_nodes))
                        li = jnp.where(hit & (colf < n_nodes), 1, 0
                                       ).astype(jnp.int32)
                        dbuf[a, pl.ds(c * 16, 16)] = li
                    return rcarry

                lax.fori_loop(0, _B, row_step, 0, unroll=2)
                pltpu.sync_copy(dbuf, out_hbm.at[pl.ds(r0, _B), pl.ds(c0, _B)])

        return carry

    lax.fori_loop(0, qmax, pair_step, 0)


def kernel(N_values, N_probs, r_probs, u_cat, u_links):
    mesh = plsc.VectorSubcoreMesh(core_axis_name="c", subcore_axis_name="s")
    sc = pl.kernel(
        _sc_body,
        mesh=mesh,
        compiler_params=pltpu.CompilerParams(needs_layout_passes=False),
        out_type=jax.ShapeDtypeStruct((_N, _N), jnp.int32),
        scratch_types=[
            pltpu.VMEM((16,), jnp.float32),
            pltpu.VMEM((16,), jnp.float32),
            pltpu.VMEM((16,), jnp.float32),
            pltpu.VMEM((16,), jnp.float32),
            pltpu.VMEM((_B, _B), jnp.float32),
            pltpu.VMEM((_B, _B), jnp.int32),
            pltpu.VMEM((_B, _B), jnp.int32),
        ],
    )
    return sc(
        N_values,
        N_probs,
        r_probs,
        jnp.broadcast_to(u_cat, (16,)),
        u_links,
    )


# speedup vs baseline: 13.0943x; 8.0235x over previous
import jax
import jax.numpy as jnp
import numpy as np
from jax.experimental import pallas as pl
from jax.experimental.pallas import tpu as pltpu

_BLK = 1024
_NBUF = 3


def _sample(nv_ref, np_ref, rp_ref, uc_ref):
    k_sizes = np_ref.shape[1]
    total = np_ref[0, 0]
    for k in range(1, k_sizes):
        total = total + np_ref[0, k]
    u = uc_ref[0]
    idx = 0
    csums = []
    acc = None
    for k in range(k_sizes):
        p = np_ref[0, k] / total
        acc = p if acc is None else acc + p
        csums.append(acc)
    for k in range(k_sizes - 1, -1, -1):
        idx = jnp.where(csums[k] >= u, k, idx)
    return nv_ref[0, idx], rp_ref[0, idx]


def _make_body(num_steps, blk):
    def _body(bi_ref, bj_ref, nv_ref, np_ref, rp_ref, uc_ref, u_hbm, out_ref,
              ubuf, obuf, tbuf, zbuf, isems, osems):
        k = pl.program_id(0)
        slot = jax.lax.rem(k, _NBUF)
        islot = jax.lax.rem(k, 2)
        bi = bi_ref[k]
        bj = bj_ref[k]

        n_nodes, r = _sample(nv_ref, np_ref, rp_ref, uc_ref)
        nf = n_nodes

        def needed(step):
            pb = bi_ref[step]
            qb = bj_ref[step]
            return ((pb * blk).astype(jnp.float32) < nf) & (
                (qb * blk).astype(jnp.float32) < nf)

        def start_in(step, s):
            pltpu.make_async_copy(
                u_hbm.at[pl.ds(bi_ref[step] * blk, blk),
                         pl.ds(bj_ref[step] * blk, blk)],
                ubuf.at[s],
                isems.at[s],
            ).start()

        @pl.when(k == 0)
        def _():
            zbuf[...] = jnp.zeros_like(zbuf)

            @pl.when(needed(0))
            def _():
                start_in(0, islot)

        @pl.when(k + 1 < num_steps)
        def _():
            @pl.when(needed(k + 1))
            def _():
                start_in(k + 1, jax.lax.rem(k + 1, 2))

        def _wait_step(step, wslot):
            pb = bi_ref[step]
            qb = bj_ref[step]
            pltpu.make_async_copy(
                obuf.at[wslot],
                out_ref.at[pl.ds(pb * blk, blk), pl.ds(qb * blk, blk)],
                osems.at[wslot, 0],
            ).wait()

            @pl.when(pb != qb)
            def _():
                pltpu.make_async_copy(
                    tbuf.at[wslot],
                    out_ref.at[pl.ds(qb * blk, blk), pl.ds(pb * blk, blk)],
                    osems.at[wslot, 1],
                ).wait()

        @pl.when(k >= _NBUF)
        def _():
            _wait_step(k - _NBUF, slot)

        blk_needed = needed(k)

        @pl.when(blk_needed)
        def _():
            pltpu.make_async_copy(
                u_hbm.at[pl.ds(bi * blk, blk), pl.ds(bj * blk, blk)],
                ubuf.at[islot],
                isems.at[islot],
            ).wait()

            row = (bi * blk
                   + jax.lax.broadcasted_iota(jnp.int32, (blk, blk), 0)
                   ).astype(jnp.float32)
            col = (bj * blk
                   + jax.lax.broadcasted_iota(jnp.int32, (blk, blk), 1)
                   ).astype(jnp.float32)
            u = ubuf[islot]
            lu = ((u <= r) & (col > row) & (row < nf) & (col < nf)
                  ).astype(jnp.int32)
            lt = lu.T

            @pl.when(bi == bj)
            def _():
                obuf[slot] = lu | lt

            @pl.when(bi != bj)
            def _():
                obuf[slot] = lu
                tbuf[slot] = lt

            pltpu.make_async_copy(
                obuf.at[slot],
                out_ref.at[pl.ds(bi * blk, blk), pl.ds(bj * blk, blk)],
                osems.at[slot, 0],
            ).start()

            @pl.when(bi != bj)
            def _():
                pltpu.make_async_copy(
                    tbuf.at[slot],
                    out_ref.at[pl.ds(bj * blk, blk), pl.ds(bi * blk, blk)],
                    osems.at[slot, 1],
                ).start()

        @pl.when(jnp.logical_not(blk_needed))
        def _():
            pltpu.make_async_copy(
                zbuf,
                out_ref.at[pl.ds(bi * blk, blk), pl.ds(bj * blk, blk)],
                osems.at[slot, 0],
            ).start()

            @pl.when(bi != bj)
            def _():
                pltpu.make_async_copy(
                    zbuf,
                    out_ref.at[pl.ds(bj * blk, blk), pl.ds(bi * blk, blk)],
                    osems.at[slot, 1],
                ).start()

        @pl.when(k == num_steps - 1)
        def _():
            for s in range(max(0, num_steps - _NBUF), num_steps):
                _wait_step(s, s % _NBUF)

    return _body


def kernel(N_values, N_probs, r_probs, u_cat, u_links):
    n = u_links.shape[0]
    g = n // _BLK
    pairs = [(i, j) for i in range(g) for j in range(i, g)]
    num_steps = len(pairs)
    bi_arr = jnp.asarray(np.array([p[0] for p in pairs], dtype=np.int32))
    bj_arr = jnp.asarray(np.array([p[1] for p in pairs], dtype=np.int32))

    grid_spec = pltpu.PrefetchScalarGridSpec(
        num_scalar_prefetch=6,
        grid=(num_steps,),
        in_specs=[
            pl.BlockSpec(memory_space=pl.ANY),
        ],
        out_specs=pl.BlockSpec(memory_space=pl.ANY),
        scratch_shapes=[
            pltpu.VMEM((2, _BLK, _BLK), jnp.float32),
            pltpu.VMEM((_NBUF, _BLK, _BLK), jnp.int32),
            pltpu.VMEM((_NBUF, _BLK, _BLK), jnp.int32),
            pltpu.VMEM((_BLK, _BLK), jnp.int32),
            pltpu.SemaphoreType.DMA((2,)),
            pltpu.SemaphoreType.DMA((_NBUF, 2)),
        ],
    )

    return pl.pallas_call(
        _make_body(num_steps, _BLK),
        grid_spec=grid_spec,
        out_shape=jax.ShapeDtypeStruct((n, n), jnp.int32),
    )(
        bi_arr,
        bj_arr,
        N_values.reshape(1, 16),
        N_probs.reshape(1, 16),
        r_probs.reshape(1, 16),
        u_cat.reshape(1),
        u_links,
    )
